# baseline (speedup 1.0000x reference)
import jax
import jax.numpy as jnp
from jax import lax
from jax.experimental import pallas as pl
from jax.experimental.pallas import tpu as pltpu

B, T, DM, H, D, BT = 4, 2048, 1024, 16, 64, 16
ETA_BASE = 0.01
LN_EPS = 1e-05
THETA = 10000.0
NC = T // BT


def _ttt_body(k_ref, v_ref, q_ref, e_ref, w0_ref, lnw_ref, lnb_ref,
              z_ref, wf_ref):
    lnw = lnw_ref[0]
    lnb = lnb_ref[0]
    row = lax.broadcasted_iota(jnp.int32, (BT, BT), 0)
    col = lax.broadcasted_iota(jnp.int32, (BT, BT), 1)
    mask = row > col

    def _ln_stats(a):
        mu = jnp.mean(a, axis=-1, keepdims=True)
        ac = a - mu
        var = jnp.mean(ac * ac, axis=-1, keepdims=True)
        inv = lax.rsqrt(var + LN_EPS)
        return ac * inv, inv

    def step(c, Wst):
        t0 = c * BT
        Kb = k_ref[0, pl.ds(t0, BT), :]
        Vb = v_ref[0, pl.ds(t0, BT), :]
        Qb = q_ref[0, pl.ds(t0, BT), :]
        Eb = e_ref[0, pl.ds(t0, BT), :]

        tK = lax.dot_general(Kb, Wst, (((1,), (1,)), ((), ())),
                             preferred_element_type=jnp.float32)
        xhat, inv = _ln_stats(tK)
        g = (2.0 / BT) * (Kb + (xhat * lnw + lnb) - Vb)
        dxhat = g * lnw
        u = (dxhat
             - jnp.mean(dxhat, axis=-1, keepdims=True)
             - xhat * jnp.mean(dxhat * xhat, axis=-1, keepdims=True)) * inv
        Ue = u * Eb

        S = lax.dot_general(Qb, Kb, (((1,), (1,)), ((), ())),
                            preferred_element_type=jnp.float32)
        corr = jnp.dot(jnp.where(mask, S, 0.0), Ue,
                       preferred_element_type=jnp.float32)
        TQ = lax.dot_general(Qb, Wst, (((1,), (1,)), ((), ())),
                             preferred_element_type=jnp.float32) - corr
        qhat, _ = _ln_stats(TQ)
        z_ref[0, pl.ds(t0, BT), :] = Qb + (qhat * lnw + lnb)

        return Wst - lax.dot_general(Ue, Kb, (((0,), (0,)), ((), ())),
                                     preferred_element_type=jnp.float32)

    wf_ref[0] = lax.fori_loop(0, NC, step, w0_ref[0], unroll=False)


@jax.jit
def kernel(x, W, Wq, Wk, Wv, Wo, Wlr, ln_w, ln_b):
    xf = x.reshape(B * T, DM)
    q = (xf @ Wq.T).reshape(B, T, H, D).transpose(0, 2, 1, 3)
    k = (xf @ Wk.T).reshape(B, T, H, D).transpose(0, 2, 1, 3)
    v = (xf @ Wv.T).reshape(B, T, H, D).transpose(0, 2, 1, 3)

    pos = jnp.arange(T, dtype=jnp.float32)
    inv_freq = 1.0 / (THETA ** (jnp.arange(0, D, 2, dtype=jnp.float32) / D))
    freqs = pos[:, None] * inv_freq[None, :]
    emb = jnp.concatenate([freqs, freqs], axis=-1)
    cos = jnp.cos(emb)[None, None]
    sin = jnp.sin(emb)[None, None]

    def rot(a):
        a1, a2 = jnp.split(a, 2, axis=-1)
        return jnp.concatenate([-a2, a1], axis=-1)

    q = (q * cos + rot(q) * sin).reshape(B * H, T, D)
    k = (k * cos + rot(k) * sin).reshape(B * H, T, D)
    v = v.reshape(B * H, T, D)

    eta = ETA_BASE * jax.nn.sigmoid(xf @ Wlr.T).reshape(B, T, 1)
    w0 = W.reshape(B * H, D, D)
    lnw3 = ln_w.reshape(H, 1, D)
    lnb3 = ln_b.reshape(H, 1, D)

    z, wf = pl.pallas_call(
        _ttt_body,
        grid=(B * H,),
        in_specs=[
            pl.BlockSpec((1, T, D), lambda i: (i, 0, 0)),
            pl.BlockSpec((1, T, D), lambda i: (i, 0, 0)),
            pl.BlockSpec((1, T, D), lambda i: (i, 0, 0)),
            pl.BlockSpec((1, T, 1), lambda i: (i // H, 0, 0)),
            pl.BlockSpec((1, D, D), lambda i: (i, 0, 0)),
            pl.BlockSpec((1, 1, D), lambda i: (i % H, 0, 0)),
            pl.BlockSpec((1, 1, D), lambda i: (i % H, 0, 0)),
        ],
        out_specs=[
            pl.BlockSpec((1, T, D), lambda i: (i, 0, 0)),
            pl.BlockSpec((1, D, D), lambda i: (i, 0, 0)),
        ],
        out_shape=[
            jax.ShapeDtypeStruct((B * H, T, D), jnp.float32),
            jax.ShapeDtypeStruct((B * H, D, D), jnp.float32),
        ],
        compiler_params=pltpu.CompilerParams(
            dimension_semantics=("parallel",)),
    )(k, v, q, eta, w0, lnw3, lnb3)

    zt = z.reshape(B, H, T, D).transpose(0, 2, 1, 3).reshape(B * T, DM)
    out = (zt @ Wo.T).reshape(B, T, DM)
    return out, wf.reshape(B, H, D, D)


# speedup vs baseline: 1.2732x; 1.2732x over previous
import jax
import jax.numpy as jnp
from jax import lax
from jax.experimental import pallas as pl
from jax.experimental.pallas import tpu as pltpu

B, T, DM, H, D, BT = 4, 2048, 1024, 16, 64, 16
ETA_BASE = 0.01
LN_EPS = 1e-05
THETA = 10000.0
NC = T // BT
G = (B * H) // 2


def _ttt_body(k_ref, v_ref, q_ref, e_ref, w0_ref, lnw_ref, lnb_ref,
              z_ref, wf_ref):
    c = pl.program_id(1)

    @pl.when(c == 0)
    def _init():
        wf_ref[...] = w0_ref[...]

    row = lax.broadcasted_iota(jnp.int32, (BT, BT), 0)
    col = lax.broadcasted_iota(jnp.int32, (BT, BT), 1)
    mask = row > col

    def _ln_stats(a):
        mu = jnp.mean(a, axis=-1, keepdims=True)
        ac = a - mu
        var = jnp.mean(ac * ac, axis=-1, keepdims=True)
        inv = lax.rsqrt(var + LN_EPS)
        return ac * inv, inv

    for hh in range(G):
        lnw = lnw_ref[hh % H]
        lnb = lnb_ref[hh % H]
        Wst = wf_ref[hh]
        Kb = k_ref[hh]
        Vb = v_ref[hh]
        Qb = q_ref[hh]
        Eb = e_ref[hh // H]

        tK = lax.dot_general(Kb, Wst, (((1,), (1,)), ((), ())),
                             preferred_element_type=jnp.float32)
        xhat, inv = _ln_stats(tK)
        g = (2.0 / BT) * (Kb + (xhat * lnw + lnb) - Vb)
        dxhat = g * lnw
        u = (dxhat
             - jnp.mean(dxhat, axis=-1, keepdims=True)
             - xhat * jnp.mean(dxhat * xhat, axis=-1, keepdims=True)) * inv
        Ue = u * Eb

        S = lax.dot_general(Qb, Kb, (((1,), (1,)), ((), ())),
                            preferred_element_type=jnp.float32)
        corr = jnp.dot(jnp.where(mask, S, 0.0), Ue,
                       preferred_element_type=jnp.float32)
        TQ = lax.dot_general(Qb, Wst, (((1,), (1,)), ((), ())),
                             preferred_element_type=jnp.float32) - corr
        qhat, _ = _ln_stats(TQ)
        z_ref[hh] = Qb + (qhat * lnw + lnb)

        wf_ref[hh] = Wst - lax.dot_general(Ue, Kb, (((0,), (0,)), ((), ())),
                                           preferred_element_type=jnp.float32)


@jax.jit
def kernel(x, W, Wq, Wk, Wv, Wo, Wlr, ln_w, ln_b):
    xf = x.reshape(B * T, DM)
    q = (xf @ Wq.T).reshape(B, T, H, D).transpose(0, 2, 1, 3)
    k = (xf @ Wk.T).reshape(B, T, H, D).transpose(0, 2, 1, 3)
    v = (xf @ Wv.T).reshape(B, T, H, D).transpose(0, 2, 1, 3)

    pos = jnp.arange(T, dtype=jnp.float32)
    inv_freq = 1.0 / (THETA ** (jnp.arange(0, D, 2, dtype=jnp.float32) / D))
    freqs = pos[:, None] * inv_freq[None, :]
    emb = jnp.concatenate([freqs, freqs], axis=-1)
    cos = jnp.cos(emb)[None, None]
    sin = jnp.sin(emb)[None, None]

    def rot(a):
        a1, a2 = jnp.split(a, 2, axis=-1)
        return jnp.concatenate([-a2, a1], axis=-1)

    q = (q * cos + rot(q) * sin).reshape(B * H, T, D)
    k = (k * cos + rot(k) * sin).reshape(B * H, T, D)
    v = v.reshape(B * H, T, D)

    eta = ETA_BASE * jax.nn.sigmoid(xf @ Wlr.T).reshape(B, T, 1)
    w0 = W.reshape(B * H, D, D)
    lnw3 = ln_w.reshape(H, 1, D)
    lnb3 = ln_b.reshape(H, 1, D)

    z, wf = pl.pallas_call(
        _ttt_body,
        grid=(2, NC),
        in_specs=[
            pl.BlockSpec((G, BT, D), lambda i, c: (i, c, 0)),
            pl.BlockSpec((G, BT, D), lambda i, c: (i, c, 0)),
            pl.BlockSpec((G, BT, D), lambda i, c: (i, c, 0)),
            pl.BlockSpec((B // 2, BT, 1), lambda i, c: (i, c, 0)),
            pl.BlockSpec((G, D, D), lambda i, c: (i, 0, 0)),
            pl.BlockSpec((H, 1, D), lambda i, c: (0, 0, 0)),
            pl.BlockSpec((H, 1, D), lambda i, c: (0, 0, 0)),
        ],
        out_specs=[
            pl.BlockSpec((G, BT, D), lambda i, c: (i, c, 0)),
            pl.BlockSpec((G, D, D), lambda i, c: (i, 0, 0)),
        ],
        out_shape=[
            jax.ShapeDtypeStruct((B * H, T, D), jnp.float32),
            jax.ShapeDtypeStruct((B * H, D, D), jnp.float32),
        ],
        compiler_params=pltpu.CompilerParams(
            dimension_semantics=("parallel", "arbitrary")),
    )(k, v, q, eta, w0, lnw3, lnb3)

    zt = z.reshape(B, H, T, D).transpose(0, 2, 1, 3).reshape(B * T, DM)
    out = (zt @ Wo.T).reshape(B, T, DM)
    return out, wf.reshape(B, H, D, D)


# speedup vs baseline: 4.1483x; 3.2582x over previous
import jax
import jax.numpy as jnp
from jax import lax
from jax.experimental import pallas as pl
from jax.experimental.pallas import tpu as pltpu

B, T, DM, H, D, BT = 4, 2048, 1024, 16, 64, 16
ETA_BASE = 0.01
LN_EPS = 1e-05
THETA = 10000.0
NC = T // BT
G = (B * H) // 2


def _ttt_body(k_ref, v_ref, q_ref, e_ref, w0_ref, lnw_ref, lnb_ref,
              z_ref, wf_ref, tk_s, ue_s, tq_s):
    c = pl.program_id(1)

    @pl.when(c == 0)
    def _init():
        wf_ref[...] = w0_ref[...]

    row = lax.broadcasted_iota(jnp.int32, (BT, BT), 0)
    col = lax.broadcasted_iota(jnp.int32, (BT, BT), 1)
    mask = row > col

    def _ln_stats(a):
        mu = jnp.mean(a, axis=-1, keepdims=True)
        ac = a - mu
        var = jnp.mean(ac * ac, axis=-1, keepdims=True)
        inv = lax.rsqrt(var + LN_EPS)
        return ac * inv, inv

    for hh in range(G):
        tk_s[pl.ds(hh * BT, BT), :] = lax.dot_general(
            k_ref[hh], wf_ref[hh], (((1,), (1,)), ((), ())),
            preferred_element_type=jnp.float32)

    lnw = lnw_ref[...]
    lnb = lnb_ref[...]
    Kall = k_ref[...].reshape(G * BT, D)
    Vall = v_ref[...].reshape(G * BT, D)
    tK = tk_s[...]
    xhat, inv = _ln_stats(tK)
    g = (2.0 / BT) * (Kall + (xhat * lnw + lnb) - Vall)
    dxhat = g * lnw
    u = (dxhat
         - jnp.mean(dxhat, axis=-1, keepdims=True)
         - xhat * jnp.mean(dxhat * xhat, axis=-1, keepdims=True)) * inv
    ue_s[...] = u * e_ref[0, 0]

    for hh in range(G):
        Qb = q_ref[hh]
        S = lax.dot_general(Qb, k_ref[hh], (((1,), (1,)), ((), ())),
                            preferred_element_type=jnp.float32)
        corr = jnp.dot(jnp.where(mask, S, 0.0), ue_s[pl.ds(hh * BT, BT), :],
                       preferred_element_type=jnp.float32)
        tq_s[pl.ds(hh * BT, BT), :] = lax.dot_general(
            Qb, wf_ref[hh], (((1,), (1,)), ((), ())),
            preferred_element_type=jnp.float32) - corr

    qhat, _ = _ln_stats(tq_s[...])
    Qall = q_ref[...].reshape(G * BT, D)
    z_ref[...] = (Qall + (qhat * lnw + lnb)).reshape(G, BT, D)

    for hh in range(G):
        wf_ref[hh] = wf_ref[hh] - lax.dot_general(
            ue_s[pl.ds(hh * BT, BT), :], k_ref[hh], (((0,), (0,)), ((), ())),
            preferred_element_type=jnp.float32)


@jax.jit
def kernel(x, W, Wq, Wk, Wv, Wo, Wlr, ln_w, ln_b):
    xf = x.reshape(B * T, DM)
    q = (xf @ Wq.T).reshape(B, T, H, D).transpose(0, 2, 1, 3)
    k = (xf @ Wk.T).reshape(B, T, H, D).transpose(0, 2, 1, 3)
    v = (xf @ Wv.T).reshape(B, T, H, D).transpose(0, 2, 1, 3)

    pos = jnp.arange(T, dtype=jnp.float32)
    inv_freq = 1.0 / (THETA ** (jnp.arange(0, D, 2, dtype=jnp.float32) / D))
    freqs = pos[:, None] * inv_freq[None, :]
    emb = jnp.concatenate([freqs, freqs], axis=-1)
    cos = jnp.cos(emb)[None, None]
    sin = jnp.sin(emb)[None, None]

    def rot(a):
        a1, a2 = jnp.split(a, 2, axis=-1)
        return jnp.concatenate([-a2, a1], axis=-1)

    q = (q * cos + rot(q) * sin).reshape(B * H, T, D)
    k = (k * cos + rot(k) * sin).reshape(B * H, T, D)
    v = v.reshape(B * H, T, D)

    et = (ETA_BASE * jax.nn.sigmoid(xf @ Wlr.T)).reshape(2, 2, NC, 1, BT)
    e4 = jnp.broadcast_to(et, (2, 2, NC, H, BT)).transpose(0, 2, 1, 3, 4)
    e4 = jnp.broadcast_to(e4.reshape(2, NC, G * BT, 1), (2, NC, G * BT, D))

    w0 = W.reshape(B * H, D, D)
    lnw_big = jnp.tile(jnp.repeat(ln_w, BT, axis=0), (G // H, 1))
    lnb_big = jnp.tile(jnp.repeat(ln_b, BT, axis=0), (G // H, 1))

    z, wf = pl.pallas_call(
        _ttt_body,
        grid=(2, NC),
        in_specs=[
            pl.BlockSpec((G, BT, D), lambda i, c: (i, c, 0)),
            pl.BlockSpec((G, BT, D), lambda i, c: (i, c, 0)),
            pl.BlockSpec((G, BT, D), lambda i, c: (i, c, 0)),
            pl.BlockSpec((1, 1, G * BT, D), lambda i, c: (i, c, 0, 0)),
            pl.BlockSpec((G, D, D), lambda i, c: (i, 0, 0)),
            pl.BlockSpec((G * BT, D), lambda i, c: (0, 0)),
            pl.BlockSpec((G * BT, D), lambda i, c: (0, 0)),
        ],
        out_specs=[
            pl.BlockSpec((G, BT, D), lambda i, c: (i, c, 0)),
            pl.BlockSpec((G, D, D), lambda i, c: (i, 0, 0)),
        ],
        out_shape=[
            jax.ShapeDtypeStruct((B * H, T, D), jnp.float32),
            jax.ShapeDtypeStruct((B * H, D, D), jnp.float32),
        ],
        scratch_shapes=[
            pltpu.VMEM((G * BT, D), jnp.float32),
            pltpu.VMEM((G * BT, D), jnp.float32),
            pltpu.VMEM((G * BT, D), jnp.float32),
        ],
        compiler_params=pltpu.CompilerParams(
            dimension_semantics=("parallel", "arbitrary")),
    )(k, v, q, e4, w0, lnw_big, lnb_big)

    zt = z.reshape(B, H, T, D).transpose(0, 2, 1, 3).reshape(B * T, DM)
    out = (zt @ Wo.T).reshape(B, T, DM)
    return out, wf.reshape(B, H, D, D)
